# baseline (speedup 1.0000x reference)
import functools

import jax
import jax.numpy as jnp
from jax import lax
from jax.experimental import pallas as pl
from jax.experimental.pallas import tpu as pltpu
from jax.experimental.pallas import tpu_sc as plsc

B = 16384
D = 1536
V = 32
L = 16
NC = 2
NS = 16
NW = NC * NS
B_PER_W = B // NW
CHUNK = 16
NCHUNK = B_PER_W // CHUNK


def _gather_body(table_hbm, idx_hbm, out_hbm, table_v, idx_v, bufs, ssem):
    sid = lax.axis_index("s")
    wid = sid * NC + lax.axis_index("c")
    base = wid * B_PER_W

    pltpu.sync_copy(table_hbm, table_v)
    pltpu.sync_copy(idx_hbm.at[pl.ds(base, B_PER_W)], idx_v)

    lanes = lax.iota(jnp.int32, L)

    def out_slice(g):
        return out_hbm.at[pl.ds(base + g * CHUNK, CHUNK)]

    def build_row(g, cur, r):
        pos = jnp.broadcast_to(g * CHUNK + r, (L,)).astype(jnp.int32)
        row_ids = plsc.load_gather(idx_v, [pos])
        for j in range(D // L):
            cols = lanes + (j * L)
            bufs[cur, r, pl.ds(j * L, L)] = plsc.load_gather(table_v, [row_ids, cols])

    def loop_body(g2, _):
        for cur in (0, 1):
            g = g2 * 2 + cur
            @pl.when(g2 > 0)
            def _():
                pltpu.make_async_copy(bufs.at[cur], out_slice(g - 2), ssem).wait()

            def row_body(r, carry):
                build_row(g, cur, r)
                return carry

            lax.fori_loop(0, CHUNK, row_body, 0)
            pltpu.async_copy(bufs.at[cur], out_slice(g), ssem)
        return _

    lax.fori_loop(0, NCHUNK // 2, loop_body, None)

    pltpu.make_async_copy(bufs.at[0], out_slice(NCHUNK - 2), ssem).wait()
    pltpu.make_async_copy(bufs.at[1], out_slice(NCHUNK - 1), ssem).wait()


def kernel(b, cat_ids):
    cat_ids = cat_ids.astype(jnp.int32)
    mesh = plsc.VectorSubcoreMesh(core_axis_name="c", subcore_axis_name="s")
    run = functools.partial(
        pl.kernel,
        mesh=mesh,
        compiler_params=pltpu.CompilerParams(needs_layout_passes=False),
        out_type=jax.ShapeDtypeStruct((B, D), jnp.float32),
        scratch_types=[
            pltpu.VMEM((V, D), jnp.float32),
            pltpu.VMEM((B_PER_W,), jnp.int32),
            pltpu.VMEM((2, CHUNK, D), jnp.float32),
            pltpu.SemaphoreType.DMA,
        ],
    )(_gather_body)
    return run(b, cat_ids)


# speedup vs baseline: 2.2658x; 2.2658x over previous
import functools

import jax
import jax.numpy as jnp
from jax import lax
from jax.experimental import pallas as pl
from jax.experimental.pallas import tpu as pltpu
from jax.experimental.pallas import tpu_sc as plsc

B = 16384
D = 1536
V = 32
L = 16
NC = 2
NS = 16
NW = NC * NS
B_PER_W = B // NW
CHUNK = 16
NCHUNK = B_PER_W // CHUNK


def _gather_body(table_hbm, idx_hbm, out_hbm, rep_hbm, table_v, idx_v, bufs, gsem, ssem):
    sid = lax.axis_index("s")
    wid = sid * NC + lax.axis_index("c")
    base = wid * B_PER_W

    pltpu.sync_copy(table_hbm, table_v)
    pltpu.sync_copy(table_v, rep_hbm.at[pl.ds(wid * V, V)])
    pltpu.sync_copy(idx_hbm.at[pl.ds(base, B_PER_W)], idx_v)
    rebase = jnp.broadcast_to(wid * V, (L,)).astype(jnp.int32)
    for k in range(B_PER_W // L):
        idx_v[pl.ds(k * L, L)] = idx_v[pl.ds(k * L, L)] + rebase

    def idx_slice(g):
        return idx_v.at[pl.ds(g * CHUNK, CHUNK)]

    def out_slice(g):
        return out_hbm.at[pl.ds(base + g * CHUNK, CHUNK)]

    pltpu.async_copy(rep_hbm.at[idx_slice(0)], bufs.at[0], gsem)

    for g in range(NCHUNK):
        cur = g % 2
        nxt = (g + 1) % 2
        pltpu.make_async_copy(rep_hbm.at[idx_slice(g)], bufs.at[cur], gsem).wait()
        if g >= 1:
            pltpu.make_async_copy(bufs.at[nxt], out_slice(g - 1), ssem).wait()
        if g + 1 < NCHUNK:
            pltpu.async_copy(rep_hbm.at[idx_slice(g + 1)], bufs.at[nxt], gsem)
        pltpu.async_copy(bufs.at[cur], out_slice(g), ssem)

    pltpu.make_async_copy(bufs.at[(NCHUNK - 1) % 2], out_slice(NCHUNK - 1), ssem).wait()


def kernel(b, cat_ids):
    cat_ids = cat_ids.astype(jnp.int32)
    mesh = plsc.VectorSubcoreMesh(core_axis_name="c", subcore_axis_name="s")
    run = functools.partial(
        pl.kernel,
        mesh=mesh,
        compiler_params=pltpu.CompilerParams(needs_layout_passes=False),
        out_type=jax.ShapeDtypeStruct((B, D), jnp.float32),
        scratch_types=[
            pltpu.MemorySpace.HBM((NW * V, D), jnp.float32),
            pltpu.VMEM((V, D), jnp.float32),
            pltpu.VMEM((B_PER_W,), jnp.int32),
            pltpu.VMEM((2, CHUNK, D), jnp.float32),
            pltpu.SemaphoreType.DMA,
            pltpu.SemaphoreType.DMA,
        ],
    )(_gather_body)
    return run(b, cat_ids)


# speedup vs baseline: 2.4848x; 1.0966x over previous
import functools

import jax
import jax.numpy as jnp
from jax import lax
from jax.experimental import pallas as pl
from jax.experimental.pallas import tpu as pltpu
from jax.experimental.pallas import tpu_sc as plsc

B = 16384
D = 1536
V = 32
L = 16
NC = 2
NS = 16
NW = NC * NS
B_PER_W = B // NW
CHUNK = 16
NCHUNK = B_PER_W // CHUNK


def _gather_body(table_hbm, idx_hbm, out_hbm, rep_hbm, table_v, idx_v, bufs, gsem, ssem):
    sid = lax.axis_index("s")
    wid = sid * NC + lax.axis_index("c")
    base = wid * B_PER_W

    pltpu.sync_copy(table_hbm, table_v)
    pltpu.sync_copy(table_v, rep_hbm.at[pl.ds(wid * V, V)])
    pltpu.sync_copy(idx_hbm.at[pl.ds(base, B_PER_W)], idx_v)
    rebase = jnp.broadcast_to(wid * V, (L,)).astype(jnp.int32)
    for k in range(B_PER_W // L):
        idx_v[pl.ds(k * L, L)] = idx_v[pl.ds(k * L, L)] + rebase

    def idx_slice(g):
        return idx_v.at[pl.ds(g * CHUNK, CHUNK)]

    def out_slice(g):
        return out_hbm.at[pl.ds(base + g * CHUNK, CHUNK)]

    pltpu.async_copy(rep_hbm.at[idx_slice(0)], bufs.at[0], gsem)
    pltpu.async_copy(rep_hbm.at[idx_slice(1)], bufs.at[1], gsem)

    for g in range(NCHUNK):
        cur = g % 3
        pltpu.make_async_copy(rep_hbm.at[idx_slice(g)], bufs.at[cur], gsem).wait()
        if g >= 1:
            pltpu.make_async_copy(bufs.at[(g - 1) % 3], out_slice(g - 1), ssem).wait()
        if g + 2 < NCHUNK:
            pltpu.async_copy(rep_hbm.at[idx_slice(g + 2)], bufs.at[(g + 2) % 3], gsem)
        pltpu.async_copy(bufs.at[cur], out_slice(g), ssem)

    pltpu.make_async_copy(bufs.at[(NCHUNK - 1) % 3], out_slice(NCHUNK - 1), ssem).wait()


def kernel(b, cat_ids):
    cat_ids = cat_ids.astype(jnp.int32)
    mesh = plsc.VectorSubcoreMesh(core_axis_name="c", subcore_axis_name="s")
    run = functools.partial(
        pl.kernel,
        mesh=mesh,
        compiler_params=pltpu.CompilerParams(needs_layout_passes=False),
        out_type=jax.ShapeDtypeStruct((B, D), jnp.float32),
        scratch_types=[
            pltpu.MemorySpace.HBM((NW * V, D), jnp.float32),
            pltpu.VMEM((V, D), jnp.float32),
            pltpu.VMEM((B_PER_W,), jnp.int32),
            pltpu.VMEM((3, CHUNK, D), jnp.float32),
            pltpu.SemaphoreType.DMA,
            pltpu.SemaphoreType.DMA,
        ],
    )(_gather_body)
    return run(b, cat_ids)


# speedup vs baseline: 2.7013x; 1.0871x over previous
import functools

import jax
import jax.numpy as jnp
from jax import lax
from jax.experimental import pallas as pl
from jax.experimental.pallas import tpu as pltpu
from jax.experimental.pallas import tpu_sc as plsc

B = 16384
D = 1536
V = 32
L = 16
NC = 2
NS = 16
NW = NC * NS
B_PER_W = B // NW
CHUNK = 16
NCHUNK = B_PER_W // CHUNK


def _gather_body(rep_hbm, idx_hbm, out_hbm, idx_v, bufs, gsem, ssem):
    sid = lax.axis_index("s")
    wid = sid * NC + lax.axis_index("c")
    base = wid * B_PER_W

    pltpu.sync_copy(idx_hbm.at[pl.ds(base, B_PER_W)], idx_v)
    rebase = jnp.broadcast_to(wid * V, (L,)).astype(jnp.int32)
    for k in range(B_PER_W // L):
        idx_v[pl.ds(k * L, L)] = idx_v[pl.ds(k * L, L)] + rebase

    def idx_slice(g):
        return idx_v.at[pl.ds(g * CHUNK, CHUNK)]

    def out_slice(g):
        return out_hbm.at[pl.ds(base + g * CHUNK, CHUNK)]

    pltpu.async_copy(rep_hbm.at[idx_slice(0)], bufs.at[0], gsem)
    pltpu.async_copy(rep_hbm.at[idx_slice(1)], bufs.at[1], gsem)

    for g in range(NCHUNK):
        cur = g % 3
        pltpu.make_async_copy(rep_hbm.at[idx_slice(g)], bufs.at[cur], gsem).wait()
        if g >= 1:
            pltpu.make_async_copy(bufs.at[(g - 1) % 3], out_slice(g - 1), ssem).wait()
        if g + 2 < NCHUNK:
            pltpu.async_copy(rep_hbm.at[idx_slice(g + 2)], bufs.at[(g + 2) % 3], gsem)
        pltpu.async_copy(bufs.at[cur], out_slice(g), ssem)

    pltpu.make_async_copy(bufs.at[(NCHUNK - 1) % 3], out_slice(NCHUNK - 1), ssem).wait()


def kernel(b, cat_ids):
    cat_ids = cat_ids.astype(jnp.int32)
    rep = jnp.tile(b, (NW, 1))
    mesh = plsc.VectorSubcoreMesh(core_axis_name="c", subcore_axis_name="s")
    run = functools.partial(
        pl.kernel,
        mesh=mesh,
        compiler_params=pltpu.CompilerParams(needs_layout_passes=False),
        out_type=jax.ShapeDtypeStruct((B, D), jnp.float32),
        scratch_types=[
            pltpu.VMEM((B_PER_W,), jnp.int32),
            pltpu.VMEM((3, CHUNK, D), jnp.float32),
            pltpu.SemaphoreType.DMA,
            pltpu.SemaphoreType.DMA,
        ],
    )(_gather_body)
    return run(rep, cat_ids)
